# baseline (speedup 1.0000x reference)
import functools

import jax
import jax.numpy as jnp
from jax import lax
from jax.experimental import pallas as pl
from jax.experimental.pallas import tpu as pltpu
from jax.experimental.pallas import tpu_sc as plsc

HID = 2048
NE = 16
NT = 16384
TOPK = 2
TM = 512

NC = 2
NS = 16
NW = NC * NS
TPW = NT // NW
LANES = 16
NG = TPW // LANES
WPM = TM // TPW


def _gate_body(x_ref, w_ref, b_ref, lgt_ref, wt_ref, it_ref, lgtc_ref):
    acct = lax.dot_general(
        w_ref[...], x_ref[...], (((1,), (1,)), ((), ())),
        preferred_element_type=jnp.float32)
    acct = acct + b_ref[...].reshape(NE, 1)
    lgt_ref[...] = acct
    for w in range(WPM):
        lgtc_ref[w] = acct[:, w * TPW:(w + 1) * TPW]
    e_col = lax.broadcasted_iota(jnp.int32, (NE, TM), 0)
    m1 = jnp.max(acct, axis=0, keepdims=True)
    i1 = jnp.min(jnp.where(acct == m1, e_col, NE), axis=0, keepdims=True)
    acct2 = jnp.where(e_col == i1, -jnp.inf, acct)
    m2 = jnp.max(acct2, axis=0, keepdims=True)
    i2 = jnp.min(jnp.where(acct2 == m2, e_col, NE), axis=0, keepdims=True)
    r = jnp.exp(m2 - m1)
    den = 1.0 + r
    wt_ref[...] = jnp.concatenate([1.0 / den, r / den], axis=0)
    it_ref[...] = jnp.concatenate([i1, i2], axis=0)


def _gate_topk(x, W, b2):
    return pl.pallas_call(
        _gate_body,
        grid=(NT // TM,),
        in_specs=[
            pl.BlockSpec((TM, HID), lambda i: (i, 0)),
            pl.BlockSpec((NE, HID), lambda i: (0, 0)),
            pl.BlockSpec((NE,), lambda i: (0,)),
        ],
        out_specs=[
            pl.BlockSpec((NE, TM), lambda i: (0, i)),
            pl.BlockSpec((TOPK, TM), lambda i: (0, i)),
            pl.BlockSpec((TOPK, TM), lambda i: (0, i)),
            pl.BlockSpec((WPM, NE, TPW), lambda i: (i, 0, 0)),
        ],
        out_shape=[
            jax.ShapeDtypeStruct((NE, NT), jnp.float32),
            jax.ShapeDtypeStruct((TOPK, NT), jnp.float32),
            jax.ShapeDtypeStruct((TOPK, NT), jnp.int32),
            jax.ShapeDtypeStruct((NW, NE, TPW), jnp.float32),
        ],
        compiler_params=pltpu.CompilerParams(
            dimension_semantics=("arbitrary",)),
    )(x, W, b2)


def _combine(av, ai, bv, bi):
    take = bv > av
    return jnp.where(take, bv, av), jnp.where(take, bi, ai)


def _tree_max(vals, idxs):
    while len(vals) > 1:
        nv, ni = [], []
        for j in range(0, len(vals), 2):
            v, i = _combine(vals[j], idxs[j], vals[j + 1], idxs[j + 1])
            nv.append(v)
            ni.append(i)
        vals, idxs = nv, ni
    return vals[0], idxs[0]


@functools.partial(
    pl.kernel,
    mesh=plsc.VectorSubcoreMesh(core_axis_name="c", subcore_axis_name="s"),
    out_type=jax.ShapeDtypeStruct((NE, TOPK, NT), jnp.int32),
    scratch_types=[
        pltpu.VMEM((NE, TPW), jnp.float32),
        pltpu.VMEM((NE, TOPK, TPW), jnp.int32),
    ],
    compiler_params=pltpu.CompilerParams(needs_layout_passes=False),
)
def _mask_route(lgt_hbm, m_hbm, lgt_v, m_v):
    c = lax.axis_index("c")
    s = lax.axis_index("s")
    wid = s * NC + c
    base = wid * TPW
    pltpu.sync_copy(lgt_hbm.at[wid], lgt_v)

    def group(g, carry):
        t0 = g * LANES
        vs = [lgt_v[e, pl.ds(t0, LANES)] for e in range(NE)]
        eidx = [jnp.full((LANES,), e, jnp.int32) for e in range(NE)]
        m1, i1 = _tree_max(list(vs), list(eidx))
        neg = jnp.full((LANES,), -jnp.inf, jnp.float32)
        vs2 = [jnp.where(i1 == e, neg, vs[e]) for e in range(NE)]
        m2, i2 = _tree_max(vs2, list(eidx))
        one = jnp.full((LANES,), 1, jnp.int32)
        zero = jnp.zeros((LANES,), jnp.int32)
        for e in range(NE):
            m_v[e, 0, pl.ds(t0, LANES)] = jnp.where(i1 == e, one, zero)
            m_v[e, 1, pl.ds(t0, LANES)] = jnp.where(i2 == e, one, zero)
        return carry

    lax.fori_loop(0, NG, group, 0)
    pltpu.sync_copy(m_v, m_hbm.at[:, :, pl.ds(base, TPW)])


def kernel(x, W, b):
    logits_t, weights_t, indices_t, logits_t_chunks = _gate_topk(
        x, W, b)
    mask = _mask_route(logits_t_chunks)
    return (logits_t.T, weights_t.T, indices_t.T, mask)


# speedup vs baseline: 1.1541x; 1.1541x over previous
import functools

import jax
import jax.numpy as jnp
from jax import lax
from jax.experimental import pallas as pl
from jax.experimental.pallas import tpu as pltpu
from jax.experimental.pallas import tpu_sc as plsc

HID = 2048
NE = 16
NT = 16384
TOPK = 2
TM = 1024

NC = 2
NS = 16
NW = NC * NS
TPW = NT // NW
LANES = 16
NG = TPW // LANES
WPM = TM // TPW


def _gate_body(x_ref, w_ref, b_ref, lgt_ref, wt_ref, it_ref, lgtc_ref):
    acct = lax.dot_general(
        w_ref[...], x_ref[...], (((1,), (1,)), ((), ())),
        preferred_element_type=jnp.float32)
    acct = acct + b_ref[...].reshape(NE, 1)
    lgt_ref[...] = acct
    for w in range(WPM):
        lgtc_ref[w] = acct[:, w * TPW:(w + 1) * TPW]
    e_col = lax.broadcasted_iota(jnp.int32, (NE, TM), 0)
    m1 = jnp.max(acct, axis=0, keepdims=True)
    i1 = jnp.min(jnp.where(acct == m1, e_col, NE), axis=0, keepdims=True)
    acct2 = jnp.where(e_col == i1, -jnp.inf, acct)
    m2 = jnp.max(acct2, axis=0, keepdims=True)
    i2 = jnp.min(jnp.where(acct2 == m2, e_col, NE), axis=0, keepdims=True)
    r = jnp.exp(m2 - m1)
    den = 1.0 + r
    wt_ref[...] = jnp.concatenate([1.0 / den, r / den], axis=0)
    it_ref[...] = jnp.concatenate([i1, i2], axis=0)


def _gate_topk(x, W, b2):
    return pl.pallas_call(
        _gate_body,
        grid=(NT // TM,),
        in_specs=[
            pl.BlockSpec((TM, HID), lambda i: (i, 0)),
            pl.BlockSpec((NE, HID), lambda i: (0, 0)),
            pl.BlockSpec((NE,), lambda i: (0,)),
        ],
        out_specs=[
            pl.BlockSpec((NE, TM), lambda i: (0, i)),
            pl.BlockSpec((TOPK, TM), lambda i: (0, i)),
            pl.BlockSpec((TOPK, TM), lambda i: (0, i)),
            pl.BlockSpec((WPM, NE, TPW), lambda i: (i, 0, 0)),
        ],
        out_shape=[
            jax.ShapeDtypeStruct((NE, NT), jnp.float32),
            jax.ShapeDtypeStruct((TOPK, NT), jnp.float32),
            jax.ShapeDtypeStruct((TOPK, NT), jnp.int32),
            jax.ShapeDtypeStruct((NW, NE, TPW), jnp.float32),
        ],
        compiler_params=pltpu.CompilerParams(
            dimension_semantics=("arbitrary",)),
    )(x, W, b2)


def _combine(av, ai, bv, bi):
    take = bv > av
    return jnp.where(take, bv, av), jnp.where(take, bi, ai)


def _tree_max(vals, idxs):
    while len(vals) > 1:
        nv, ni = [], []
        for j in range(0, len(vals), 2):
            v, i = _combine(vals[j], idxs[j], vals[j + 1], idxs[j + 1])
            nv.append(v)
            ni.append(i)
        vals, idxs = nv, ni
    return vals[0], idxs[0]


@functools.partial(
    pl.kernel,
    mesh=plsc.VectorSubcoreMesh(core_axis_name="c", subcore_axis_name="s"),
    out_type=jax.ShapeDtypeStruct((NE, TOPK, NT), jnp.int32),
    scratch_types=[
        pltpu.VMEM((NE, TPW), jnp.float32),
        pltpu.VMEM((NE, TOPK, TPW), jnp.int32),
    ],
    compiler_params=pltpu.CompilerParams(needs_layout_passes=False),
)
def _mask_route(lgt_hbm, m_hbm, lgt_v, m_v):
    c = lax.axis_index("c")
    s = lax.axis_index("s")
    wid = s * NC + c
    base = wid * TPW
    pltpu.sync_copy(lgt_hbm.at[wid], lgt_v)

    def group(g, carry):
        t0 = g * LANES
        vs = [lgt_v[e, pl.ds(t0, LANES)] for e in range(NE)]
        eidx = [jnp.full((LANES,), e, jnp.int32) for e in range(NE)]
        m1, i1 = _tree_max(list(vs), list(eidx))
        neg = jnp.full((LANES,), -jnp.inf, jnp.float32)
        vs2 = [jnp.where(i1 == e, neg, vs[e]) for e in range(NE)]
        m2, i2 = _tree_max(vs2, list(eidx))
        one = jnp.full((LANES,), 1, jnp.int32)
        zero = jnp.zeros((LANES,), jnp.int32)
        for e in range(NE):
            m_v[e, 0, pl.ds(t0, LANES)] = jnp.where(i1 == e, one, zero)
            m_v[e, 1, pl.ds(t0, LANES)] = jnp.where(i2 == e, one, zero)
        return carry

    lax.fori_loop(0, NG, group, 0)
    pltpu.sync_copy(m_v, m_hbm.at[:, :, pl.ds(base, TPW)])


def kernel(x, W, b):
    logits_t, weights_t, indices_t, logits_t_chunks = _gate_topk(
        x, W, b)
    mask = _mask_route(logits_t_chunks)
    return (logits_t.T, weights_t.T, indices_t.T, mask)
